# Initial kernel scaffold; baseline (speedup 1.0000x reference)
import jax
import jax.numpy as jnp
from jax.experimental import pallas as pl


def kernel(patch):
    raise NotImplementedError("write your pallas kernel here")


# speedup vs baseline: 60.1063x; 60.1063x over previous
import math

import jax
import jax.numpy as jnp
from jax.experimental import pallas as pl
from jax.experimental.pallas import tpu as pltpu

PI = math.pi
P = 32
NPIX = P * P
NBINS = 36
EPS = 1e-8
BP = 256

ATAN_COEFFS = (
    0.9999999,
    -0.33332674,
    0.19987155,
    -0.14170083,
    0.10531722,
    -0.07302857,
    0.04057691,
    -0.014891472,
    0.0025802113,
)


def _shift_lanes(x, k):
    return pltpu.roll(x, (-k) % NPIX, axis=1)


def _dominant_orientation_kernel(x_ref, o_ref):
    x = x_ref[...].astype(jnp.bfloat16).astype(jnp.float32)

    li = jax.lax.broadcasted_iota(jnp.int32, (BP, NPIX), 1)
    j0 = (li & 31) == 0
    j31 = (li & 31) == 31
    i0 = li < P
    i31 = li >= NPIX - P

    xr = jnp.where(j31, x, _shift_lanes(x, 1))
    xl = jnp.where(j0, x, _shift_lanes(x, -1))
    t = xr - xl
    c = xl + 2.0 * x + xr

    t_u = jnp.where(i0, t, _shift_lanes(t, -P))
    t_d = jnp.where(i31, t, _shift_lanes(t, P))
    c_u = jnp.where(i0, c, _shift_lanes(c, -P))
    c_d = jnp.where(i31, c, _shift_lanes(c, P))

    gx = (t_u + 2.0 * t + t_d) * 0.125
    gy = (c_d - c_u) * 0.125

    mag = jnp.sqrt(gx * gx + gy * gy + EPS)

    xq = gx + EPS
    ax = jnp.abs(xq)
    ay = jnp.abs(gy)
    mx = jnp.maximum(ax, ay)
    mn = jnp.minimum(ax, ay)
    r = mn / jnp.maximum(mx, 1e-37)
    u = r * r
    p = jnp.float32(ATAN_COEFFS[-1])
    for coef in ATAN_COEFFS[-2::-1]:
        p = p * u + coef
    p = p * r
    p = jnp.where(ay > ax, (0.5 * PI) - p, p)
    p = jnp.where(xq < 0.0, PI - p, p)
    p = jnp.where(gy < 0.0, -p, p)

    o = p * (NBINS / (2.0 * PI)) + (NBINS * 1.5)
    bo = jnp.floor(o)
    wo1 = o - bo
    b0f = bo - NBINS
    b0f = jnp.where(b0f >= NBINS, b0f - NBINS, b0f)

    w1 = wo1 * mag
    w0 = mag - w1

    s0 = []
    s1 = []
    zero = jnp.zeros_like(w0)
    for k in range(NBINS):
        m = b0f == jnp.float32(k)
        s0.append(jnp.sum(jnp.where(m, w0, zero), axis=1, keepdims=True))
        s1.append(jnp.sum(jnp.where(m, w1, zero), axis=1, keepdims=True))
    hist = [s0[k] + s1[(k - 1) % NBINS] for k in range(NBINS)]

    sm = [
        0.33 * hist[(k - 1) % NBINS] + 0.34 * hist[k] + 0.33 * hist[(k + 1) % NBINS]
        for k in range(NBINS)
    ]

    best = sm[0]
    bidx = jnp.zeros_like(best)
    for k in range(1, NBINS):
        upd = sm[k] > best
        best = jnp.where(upd, sm[k], best)
        bidx = jnp.where(upd, jnp.float32(k), bidx)

    angle = PI - bidx * (2.0 * PI / NBINS)
    o_ref[...] = angle.reshape(1, BP, 1)


def kernel(patch):
    B = patch.shape[0]
    nb = B // BP
    x = patch.reshape(B, NPIX)
    out = pl.pallas_call(
        _dominant_orientation_kernel,
        grid=(nb,),
        in_specs=[pl.BlockSpec((BP, NPIX), lambda i: (i, 0))],
        out_specs=pl.BlockSpec((1, BP, 1), lambda i: (i, 0, 0)),
        out_shape=jax.ShapeDtypeStruct((nb, BP, 1), jnp.float32),
        compiler_params=pltpu.CompilerParams(
            dimension_semantics=("parallel",),
        ),
    )(x)
    return out.reshape(B)


# speedup vs baseline: 85.9476x; 1.4299x over previous
import math

import jax
import jax.numpy as jnp
import numpy as np
from jax.experimental import pallas as pl
from jax.experimental.pallas import tpu as pltpu
from jax.experimental.shard_map import shard_map
from jax.sharding import Mesh, NamedSharding, PartitionSpec

PI = math.pi
P = 32
NPIX = P * P
NBINS = 36
EPS = 1e-8
BP = 256

ATAN_COEFFS = (
    0.9999999,
    -0.33332674,
    0.19987155,
    -0.14170083,
    0.10531722,
    -0.07302857,
    0.04057691,
    -0.014891472,
    0.0025802113,
)


def _shift_lanes(x, k):
    return pltpu.roll(x, (-k) % NPIX, axis=1)


def _dominant_orientation_kernel(x_ref, o_ref):
    x = x_ref[...].astype(jnp.bfloat16).astype(jnp.float32)

    li = jax.lax.broadcasted_iota(jnp.int32, (BP, NPIX), 1)
    j0 = (li & 31) == 0
    j31 = (li & 31) == 31
    i0 = li < P
    i31 = li >= NPIX - P

    xr = jnp.where(j31, x, _shift_lanes(x, 1))
    xl = jnp.where(j0, x, _shift_lanes(x, -1))
    t = xr - xl
    c = xl + 2.0 * x + xr

    t_u = jnp.where(i0, t, _shift_lanes(t, -P))
    t_d = jnp.where(i31, t, _shift_lanes(t, P))
    c_u = jnp.where(i0, c, _shift_lanes(c, -P))
    c_d = jnp.where(i31, c, _shift_lanes(c, P))

    gx = (t_u + 2.0 * t + t_d) * 0.125
    gy = (c_d - c_u) * 0.125

    mag = jnp.sqrt(gx * gx + gy * gy + EPS)

    ori = jnp.arctan2(gy, gx + EPS) + (2.0 * PI)
    o = float(NBINS) * (ori + PI) / (2.0 * PI)
    bo = jnp.floor(o)
    wo1 = o - bo
    b0f = bo - NBINS
    b0f = jnp.where(b0f >= NBINS, b0f - NBINS, b0f)

    w1 = wo1 * mag
    w0 = mag - w1

    s0 = []
    s1 = []
    zero = jnp.zeros_like(w0)
    for k in range(NBINS):
        m = b0f == jnp.float32(k)
        s0.append(jnp.sum(jnp.where(m, w0, zero), axis=1, keepdims=True))
        s1.append(jnp.sum(jnp.where(m, w1, zero), axis=1, keepdims=True))
    hist = [s0[k] + s1[(k - 1) % NBINS] for k in range(NBINS)]

    sm = [
        0.33 * hist[(k - 1) % NBINS] + 0.34 * hist[k] + 0.33 * hist[(k + 1) % NBINS]
        for k in range(NBINS)
    ]

    best = sm[0]
    bidx = jnp.zeros_like(best)
    for k in range(1, NBINS):
        upd = sm[k] > best
        best = jnp.where(upd, sm[k], best)
        bidx = jnp.where(upd, jnp.float32(k), bidx)

    angle = PI - bidx * (2.0 * PI / NBINS)
    o_ref[...] = angle.reshape(1, BP, 1)


def _run_shard(x):
    b = x.shape[0]
    nb = b // BP
    out = pl.pallas_call(
        _dominant_orientation_kernel,
        grid=(nb,),
        in_specs=[pl.BlockSpec((BP, NPIX), lambda i: (i, 0))],
        out_specs=pl.BlockSpec((1, BP, 1), lambda i: (i, 0, 0)),
        out_shape=jax.ShapeDtypeStruct((nb, BP, 1), jnp.float32),
        compiler_params=pltpu.CompilerParams(
            dimension_semantics=("parallel",),
        ),
    )(x)
    return out.reshape(b)


def kernel(patch):
    B = patch.shape[0]
    x = patch.reshape(B, NPIX)
    devs = jax.devices()
    nd = 2 if len(devs) >= 2 and (B // 2) % BP == 0 else 1
    if nd == 1:
        return _run_shard(x)
    mesh = Mesh(np.asarray(devs[:nd]), ("d",))
    x = jax.lax.with_sharding_constraint(
        x, NamedSharding(mesh, PartitionSpec("d", None)))
    f = shard_map(_run_shard, mesh=mesh,
                  in_specs=PartitionSpec("d", None),
                  out_specs=PartitionSpec("d"),
                  check_rep=False)
    return f(x)


# speedup vs baseline: 86.2329x; 1.0033x over previous
import math

import jax
import jax.numpy as jnp
import numpy as np
from jax.experimental import pallas as pl
from jax.experimental.pallas import tpu as pltpu
from jax.experimental.shard_map import shard_map
from jax.sharding import Mesh, NamedSharding, PartitionSpec

PI = math.pi
P = 32
NPIX = P * P
NBINS = 36
EPS = 1e-8
BP = 256

ATAN_COEFFS = (
    0.9999999,
    -0.33332674,
    0.19987155,
    -0.14170083,
    0.10531722,
    -0.07302857,
    0.04057691,
    -0.014891472,
    0.0025802113,
)


def _shift_lanes(x, k):
    return pltpu.roll(x, (-k) % NPIX, axis=1)


def _dominant_orientation_kernel(x_ref, o_ref):
    x = x_ref[...].astype(jnp.bfloat16).astype(jnp.float32)

    li = jax.lax.broadcasted_iota(jnp.int32, (BP, NPIX), 1)
    j0 = (li & 31) == 0
    j31 = (li & 31) == 31
    i0 = li < P
    i31 = li >= NPIX - P

    xr = jnp.where(j31, x, _shift_lanes(x, 1))
    xl = jnp.where(j0, x, _shift_lanes(x, -1))
    t = xr - xl
    c = xl + 2.0 * x + xr

    t_u = jnp.where(i0, t, _shift_lanes(t, -P))
    t_d = jnp.where(i31, t, _shift_lanes(t, P))
    c_u = jnp.where(i0, c, _shift_lanes(c, -P))
    c_d = jnp.where(i31, c, _shift_lanes(c, P))

    gx = (t_u + 2.0 * t + t_d) * 0.125
    gy = (c_d - c_u) * 0.125

    mag = jnp.sqrt(gx * gx + gy * gy + EPS)

    ori = jnp.arctan2(gy, gx + EPS) + (2.0 * PI)
    o = float(NBINS) * (ori + PI) / (2.0 * PI)
    bo = jnp.floor(o)
    wo1 = o - bo
    b0f = bo - NBINS
    b0f = jnp.where(b0f >= NBINS, b0f - NBINS, b0f)

    w1 = wo1 * mag
    w0 = mag - w1

    s0 = []
    s1 = []
    for k in range(NBINS):
        mf = (b0f == jnp.float32(k)).astype(jnp.float32)
        s0.append(jnp.sum(w0 * mf, axis=1, keepdims=True))
        s1.append(jnp.sum(w1 * mf, axis=1, keepdims=True))
    hist = [s0[k] + s1[(k - 1) % NBINS] for k in range(NBINS)]

    sm = [
        0.33 * hist[(k - 1) % NBINS] + 0.34 * hist[k] + 0.33 * hist[(k + 1) % NBINS]
        for k in range(NBINS)
    ]

    best = sm[0]
    bidx = jnp.zeros_like(best)
    for k in range(1, NBINS):
        upd = sm[k] > best
        best = jnp.where(upd, sm[k], best)
        bidx = jnp.where(upd, jnp.float32(k), bidx)

    angle = PI - bidx * (2.0 * PI / NBINS)
    o_ref[...] = angle.reshape(1, BP, 1)


def _run_shard(x):
    b = x.shape[0]
    nb = b // BP
    out = pl.pallas_call(
        _dominant_orientation_kernel,
        grid=(nb,),
        in_specs=[pl.BlockSpec((BP, NPIX), lambda i: (i, 0))],
        out_specs=pl.BlockSpec((1, BP, 1), lambda i: (i, 0, 0)),
        out_shape=jax.ShapeDtypeStruct((nb, BP, 1), jnp.float32),
        compiler_params=pltpu.CompilerParams(
            dimension_semantics=("parallel",),
        ),
    )(x)
    return out.reshape(b)


def kernel(patch):
    B = patch.shape[0]
    x = patch.reshape(B, NPIX)
    devs = jax.devices()
    nd = 2 if len(devs) >= 2 and (B // 2) % BP == 0 else 1
    if nd == 1:
        return _run_shard(x)
    mesh = Mesh(np.asarray(devs[:nd]), ("d",))
    x = jax.lax.with_sharding_constraint(
        x, NamedSharding(mesh, PartitionSpec("d", None)))
    f = shard_map(_run_shard, mesh=mesh,
                  in_specs=PartitionSpec("d", None),
                  out_specs=PartitionSpec("d"),
                  check_rep=False)
    return f(x)
